# Initial kernel scaffold; baseline (speedup 1.0000x reference)
import jax
import jax.numpy as jnp
from jax.experimental import pallas as pl


def kernel(text_input, labels_input, emb_x, W_x, b_x, emb_c, W_c, b_c):
    raise NotImplementedError("write your pallas kernel here")


# speedup vs baseline: 11.9624x; 11.9624x over previous
import functools
import math

import jax
import jax.numpy as jnp
from jax import lax
from jax.experimental import pallas as pl
from jax.experimental.pallas import tpu as pltpu
from jax.experimental.pallas import tpu_sc as plsc

TEXT_VOCAB = 100000
N_CLASSES = 1000
HIDDEN = 128
BATCH = 4096
SEQ = 200

NC = 2
NS = 16
NW = NC * NS
BPW = BATCH // NW
IDX_PER_W = BPW * SEQ
LANES = 16
NH = HIDDEN // LANES
C_PAD = 1024
CPW = C_PAD // NW

_S0 = 104
_S1 = SEQ - _S0

_INV_SEQ = 1.0 / SEQ
_INV_SCALE = 1.0 / math.sqrt(float(HIDDEN))


def _bag_body(text_idx, labels_idx, emb_x, emb_c, hmean_out, hc_out,
              idx_v, buf0, buf1, acc_v, lidx_v, hcrow_v, sem0, sem1, sem2):
    wid = lax.axis_index("s") * NC + lax.axis_index("c")
    base = wid * BPW

    pltpu.sync_copy(text_idx.at[pl.ds(base * SEQ, IDX_PER_W)], idx_v)

    lbase = wid * CPW
    pltpu.sync_copy(labels_idx.at[pl.ds(lbase, CPW)], lidx_v)
    lcp = pltpu.make_async_copy(emb_c.at[lidx_v], hcrow_v, sem2)
    lcp.start()

    bufs = (buf0, buf1)
    sems = (sem0, sem1)

    def _start_gather(b, buf, sem):
        off = pl.multiple_of(b * SEQ, 8)
        pltpu.make_async_copy(
            emb_x.at[idx_v.at[pl.ds(off, _S0)]], buf.at[pl.ds(0, _S0)], sem
        ).start()
        pltpu.make_async_copy(
            emb_x.at[idx_v.at[pl.ds(off + _S0, _S1)]], buf.at[pl.ds(_S0, _S1)], sem
        ).start()

    def _wait_gather(buf, sem):
        pltpu.make_async_copy(emb_x.at[pl.ds(0, SEQ)], buf, sem).wait()

    def _reduce_store(b, buf):
        def body(i, acc):
            accs = list(acc)
            r0 = i * 8
            for rr in range(8):
                for h in range(NH):
                    accs[h] = accs[h] + buf[r0 + rr, pl.ds(h * LANES, LANES)]
            return tuple(accs)

        acc = lax.fori_loop(
            0, SEQ // 8, body,
            tuple(jnp.zeros((LANES,), jnp.float32) for _ in range(NH)))
        inv = jnp.float32(_INV_SEQ)
        for h in range(NH):
            acc_v[b, pl.ds(h * LANES, LANES)] = acc[h] * inv

    _start_gather(0, buf0, sem0)
    _start_gather(1, buf1, sem1)

    def loop_body(j, carry):
        for p in range(2):
            b = j * 2 + p
            buf, sem = bufs[p], sems[p]
            _wait_gather(buf, sem)
            _reduce_store(b, buf)

            @pl.when(b + 2 < BPW)
            def _():
                _start_gather(b + 2, buf, sem)

        return carry

    lax.fori_loop(0, BPW // 2, loop_body, 0)

    pltpu.sync_copy(acc_v, hmean_out.at[pl.ds(base, BPW)])
    lcp.wait()
    pltpu.sync_copy(hcrow_v, hc_out.at[pl.ds(lbase, CPW)])


_bag_gather = functools.partial(
    pl.kernel,
    mesh=plsc.VectorSubcoreMesh(core_axis_name="c", subcore_axis_name="s"),
    out_type=(
        jax.ShapeDtypeStruct((BATCH, HIDDEN), jnp.float32),
        jax.ShapeDtypeStruct((C_PAD, HIDDEN), jnp.float32),
    ),
    scratch_types=[
        pltpu.VMEM((IDX_PER_W,), jnp.int32),
        pltpu.VMEM((SEQ, HIDDEN), jnp.float32),
        pltpu.VMEM((SEQ, HIDDEN), jnp.float32),
        pltpu.VMEM((BPW, HIDDEN), jnp.float32),
        pltpu.VMEM((CPW,), jnp.int32),
        pltpu.VMEM((CPW, HIDDEN), jnp.float32),
        pltpu.SemaphoreType.DMA,
        pltpu.SemaphoreType.DMA,
        pltpu.SemaphoreType.DMA,
    ],
)(_bag_body)


def _dense_body(hmean_ref, wx_ref, bx_ref, hcr_ref, wc_ref, bc_ref, out_ref):
    hx = jnp.maximum(hmean_ref[...], 0.0)
    hx = lax.dot_general(hx, wx_ref[...], (((1,), (1,)), ((), ())),
                         preferred_element_type=jnp.float32) + bx_ref[...]
    hc = jnp.maximum(hcr_ref[...], 0.0)
    hc = lax.dot_general(hc, wc_ref[...], (((1,), (1,)), ((), ())),
                         preferred_element_type=jnp.float32) + bc_ref[...]
    out_ref[...] = lax.dot_general(hx, hc, (((1,), (1,)), ((), ())),
                                   preferred_element_type=jnp.float32
                                   ) * jnp.float32(_INV_SCALE)


_BB = 1024

_dense = pl.pallas_call(
    _dense_body,
    grid=(BATCH // _BB,),
    in_specs=[
        pl.BlockSpec((_BB, HIDDEN), lambda i: (i, 0)),
        pl.BlockSpec((HIDDEN, HIDDEN), lambda i: (0, 0)),
        pl.BlockSpec((1, HIDDEN), lambda i: (0, 0)),
        pl.BlockSpec((N_CLASSES, HIDDEN), lambda i: (0, 0)),
        pl.BlockSpec((HIDDEN, HIDDEN), lambda i: (0, 0)),
        pl.BlockSpec((1, HIDDEN), lambda i: (0, 0)),
    ],
    out_specs=pl.BlockSpec((_BB, N_CLASSES), lambda i: (i, 0)),
    out_shape=jax.ShapeDtypeStruct((BATCH, N_CLASSES), jnp.float32),
)


def kernel(text_input, labels_input, emb_x, W_x, b_x, emb_c, W_c, b_c):
    text_flat = text_input.reshape(-1).astype(jnp.int32)
    labels_flat = jnp.zeros((C_PAD,), jnp.int32).at[:N_CLASSES].set(
        labels_input.reshape(-1).astype(jnp.int32))
    h_mean, hc_rows = _bag_gather(text_flat, labels_flat, emb_x, emb_c)
    return _dense(h_mean, W_x, b_x.reshape(1, HIDDEN),
                  hc_rows[:N_CLASSES], W_c, b_c.reshape(1, HIDDEN))
